# Initial kernel scaffold; baseline (speedup 1.0000x reference)
import jax
import jax.numpy as jnp
from jax.experimental import pallas as pl


def kernel(x, table):
    raise NotImplementedError("write your pallas kernel here")


# speedup vs baseline: 1.8759x; 1.8759x over previous
import functools

import jax
import jax.numpy as jnp
from jax import lax
from jax.experimental import pallas as pl
from jax.experimental.pallas import tpu as pltpu
from jax.experimental.pallas import tpu_sc as plsc

NC = 2
NS = 16
NW = NC * NS
CHUNK = 128
NBUF = 8


@functools.partial(jax.jit, static_argnames=("n_chunks", "d"))
def _sc_gather(x_r, table, *, n_chunks, d):
    mesh = plsc.VectorSubcoreMesh(core_axis_name="c", subcore_axis_name="s")

    @functools.partial(
        pl.kernel,
        mesh=mesh,
        out_type=jax.ShapeDtypeStruct((NW, n_chunks, CHUNK, d), jnp.float32),
        scratch_types=[
            pltpu.VMEM((n_chunks, CHUNK), jnp.int32),
            pltpu.VMEM((NBUF, CHUNK, d), jnp.float32),
            pltpu.SemaphoreType.DMA((NBUF,)),
            pltpu.SemaphoreType.DMA((NBUF,)),
        ],
        compiler_params=pltpu.CompilerParams(use_tc_tiling_on_sc=False),
    )
    def body(x_hbm, table_hbm, out_hbm, idx_v, rows_v, in_sems, out_sems):
        wid = lax.axis_index("s") * NC + lax.axis_index("c")
        pltpu.sync_copy(x_hbm.at[wid], idx_v)

        def start_gather(slot, j):
            pltpu.async_copy(
                table_hbm.at[idx_v.at[j]], rows_v.at[slot], in_sems.at[slot]
            )

        def wait_gather(slot):
            pltpu.make_async_copy(
                table_hbm.at[pl.ds(0, CHUNK)], rows_v.at[slot], in_sems.at[slot]
            ).wait()

        def start_out(slot, j):
            pltpu.async_copy(
                rows_v.at[slot], out_hbm.at[wid, j], out_sems.at[slot]
            )

        def wait_out(slot, j):
            pltpu.make_async_copy(
                rows_v.at[slot], out_hbm.at[wid, j], out_sems.at[slot]
            ).wait()

        for b in range(NBUF):
            start_gather(b, b)

        def round_body(r, _):
            g0 = r * NBUF
            for b in range(NBUF):
                g = g0 + b
                wait_gather(b)
                start_out(b, g)
                wait_out(b, g)
                start_gather(b, g + NBUF)
            return _

        lax.fori_loop(0, n_chunks // NBUF - 1, round_body, None)

        g0 = n_chunks - NBUF
        for b in range(NBUF):
            wait_gather(b)
            start_out(b, g0 + b)
        for b in range(NBUF):
            wait_out(b, g0 + b)

    return body(x_r, table)


def kernel(x, table):
    d = table.shape[1]
    b_total = x.size
    n_chunks = b_total // (NW * CHUNK)
    x_r = x.reshape(NW, n_chunks, CHUNK).astype(jnp.int32)
    out = _sc_gather(x_r, table, n_chunks=n_chunks, d=d)
    return out.reshape(x.shape + (d,))
